# Initial kernel scaffold; baseline (speedup 1.0000x reference)
import jax
import jax.numpy as jnp
from jax.experimental import pallas as pl


def kernel(x, weight, bias, gamma, beta):
    raise NotImplementedError("write your pallas kernel here")


# speedup vs baseline: 3.8205x; 3.8205x over previous
import jax
import jax.numpy as jnp
from jax.experimental import pallas as pl
from jax.experimental.pallas import tpu as pltpu

_EPS = 1e-5


def kernel(x, weight, bias, gamma, beta):
    del bias

    N, Cin, D, H, W = x.shape
    Cout = weight.shape[1]
    HW = H * W
    DHW = D * HW
    R = 8 * Cout

    x3 = x.reshape(N, Cin, DHW).astype(jnp.float32)

    P = 2 if N % 2 == 0 else 1
    NP = N // P

    def stats_kernel(x_ref, sxx_ref, sx_ref):
        @pl.when(pl.program_id(1) == 0)
        def _init():
            sxx_ref[...] = jnp.zeros_like(sxx_ref)
            sx_ref[...] = jnp.zeros_like(sx_ref)

        xb = x_ref[...]
        sxx_ref[...] += jax.lax.dot_general(
            xb, xb, (((1,), (1,)), ((), ())),
            preferred_element_type=jnp.float32)
        sx_ref[...] += jnp.sum(xb, axis=1, keepdims=True)

    psxx, psx = pl.pallas_call(
        stats_kernel,
        out_shape=(jax.ShapeDtypeStruct((P, Cin, Cin), jnp.float32),
                   jax.ShapeDtypeStruct((P, Cin, 1), jnp.float32)),
        grid=(P, NP),
        in_specs=[pl.BlockSpec((pl.Squeezed(), Cin, DHW),
                               lambda p, i: (p * NP + i, 0, 0))],
        out_specs=(pl.BlockSpec((pl.Squeezed(), Cin, Cin),
                                lambda p, i: (p, 0, 0)),
                   pl.BlockSpec((pl.Squeezed(), Cin, 1),
                                lambda p, i: (p, 0, 0))),
        compiler_params=pltpu.CompilerParams(
            dimension_semantics=("parallel", "arbitrary")),
    )(x3)
    sxx = psxx.sum(axis=0)
    sx = psx.sum(axis=0)[:, 0]

    w_tap = jnp.transpose(weight, (2, 3, 4, 1, 0)).reshape(R, Cin)
    w_tap = w_tap.astype(jnp.float32)
    n_elem = jnp.float32(8 * N * DHW)
    sum_row = w_tap @ sx
    sumsq_row = jnp.einsum("ri,ij,rj->r", w_tap, sxx, w_tap)
    mean_c = sum_row.reshape(8, Cout).sum(axis=0) / n_elem
    var_c = sumsq_row.reshape(8, Cout).sum(axis=0) / n_elem - mean_c * mean_c
    var_c = jnp.maximum(var_c, 0.0)
    scale_c = gamma.astype(jnp.float32) * jax.lax.rsqrt(var_c + _EPS)
    shift_c = beta.astype(jnp.float32) - mean_c * scale_c

    w4 = jnp.transpose(weight, (2, 1, 3, 4, 0)).reshape(2 * Cout, 4 * Cin)
    w4 = w4.astype(jnp.float32) * jnp.tile(scale_c, 2)[:, None]
    b4 = jnp.tile(shift_c, 2).reshape(2 * Cout, 1)

    F = 4 * HW
    m_idx = jnp.arange(HW)
    lane = 4 * W * (m_idx // W) + 2 * (m_idx % W)
    e0 = (jax.nn.one_hot(lane, F, dtype=jnp.float32))

    dt = 2 if D % 2 == 0 else 1
    J = D // dt
    S = dt * HW

    def fused_kernel(w_ref, b_ref, e_ref, x_ref, o_ref, x4_ref, y4_ref):
        for dl in range(dt):
            s = jnp.dot(x_ref[:, dl * HW:(dl + 1) * HW], e_ref[...],
                        preferred_element_type=jnp.float32)
            x4_ref[0 * Cin:1 * Cin, dl * F:(dl + 1) * F] = s
            x4_ref[1 * Cin:2 * Cin, dl * F:(dl + 1) * F] = pltpu.roll(s, 1, 1)
            x4_ref[2 * Cin:3 * Cin, dl * F:(dl + 1) * F] = pltpu.roll(s, 2 * W, 1)
            x4_ref[3 * Cin:4 * Cin, dl * F:(dl + 1) * F] = pltpu.roll(s, 2 * W + 1, 1)
        y4_ref[...] = jnp.maximum(
            jnp.dot(w_ref[...], x4_ref[...],
                    preferred_element_type=jnp.float32) + b_ref[...],
            0.0)
        for dl in range(dt):
            for kd in range(2):
                o_ref[:, (2 * dl + kd) * F:(2 * dl + kd + 1) * F] = (
                    y4_ref[kd * Cout:(kd + 1) * Cout, dl * F:(dl + 1) * F])

    out3 = pl.pallas_call(
        fused_kernel,
        out_shape=jax.ShapeDtypeStruct((N, Cout, 8 * DHW), jnp.float32),
        grid=(N, J),
        in_specs=[
            pl.BlockSpec((2 * Cout, 4 * Cin), lambda n, j: (0, 0)),
            pl.BlockSpec((2 * Cout, 1), lambda n, j: (0, 0)),
            pl.BlockSpec((HW, F), lambda n, j: (0, 0)),
            pl.BlockSpec((pl.Squeezed(), Cin, S), lambda n, j: (n, 0, j)),
        ],
        out_specs=pl.BlockSpec((pl.Squeezed(), Cout, 8 * S),
                               lambda n, j: (n, 0, j)),
        scratch_shapes=[pltpu.VMEM((4 * Cin, dt * F), jnp.float32),
                        pltpu.VMEM((2 * Cout, dt * F), jnp.float32)],
        compiler_params=pltpu.CompilerParams(
            dimension_semantics=("parallel", "parallel"),
            vmem_limit_bytes=60 << 20),
    )(w4, b4, e0, x3)

    return out3.reshape(N, Cout, 2 * D, 2 * H, 2 * W)


# speedup vs baseline: 4.1958x; 1.0982x over previous
import jax
import jax.numpy as jnp
from jax.experimental import pallas as pl
from jax.experimental.pallas import tpu as pltpu

_EPS = 1e-5


def kernel(x, weight, bias, gamma, beta):
    del bias

    N, Cin, D, H, W = x.shape
    Cout = weight.shape[1]
    HW = H * W
    DHW = D * HW
    R = 8 * Cout

    x3 = x.reshape(N, Cin, DHW).astype(jnp.float32)

    P = 2 if N % 2 == 0 else 1
    NP = N // P

    def stats_kernel(x_ref, sxx_ref, sx_ref):
        @pl.when(pl.program_id(1) == 0)
        def _init():
            sxx_ref[...] = jnp.zeros_like(sxx_ref)
            sx_ref[...] = jnp.zeros_like(sx_ref)

        xb = x_ref[...]
        sxx_ref[...] += jax.lax.dot_general(
            xb, xb, (((1,), (1,)), ((), ())),
            preferred_element_type=jnp.float32)
        sx_ref[...] += jnp.sum(xb, axis=1, keepdims=True)

    psxx, psx = pl.pallas_call(
        stats_kernel,
        out_shape=(jax.ShapeDtypeStruct((P, Cin, Cin), jnp.float32),
                   jax.ShapeDtypeStruct((P, Cin, 1), jnp.float32)),
        grid=(P, NP),
        in_specs=[pl.BlockSpec((pl.Squeezed(), Cin, DHW),
                               lambda p, i: (p * NP + i, 0, 0))],
        out_specs=(pl.BlockSpec((pl.Squeezed(), Cin, Cin),
                                lambda p, i: (p, 0, 0)),
                   pl.BlockSpec((pl.Squeezed(), Cin, 1),
                                lambda p, i: (p, 0, 0))),
        compiler_params=pltpu.CompilerParams(
            dimension_semantics=("parallel", "arbitrary")),
    )(x3)
    sxx = psxx.sum(axis=0)
    sx = psx.sum(axis=0)[:, 0]

    w_tap = jnp.transpose(weight, (2, 3, 4, 1, 0)).reshape(R, Cin)
    w_tap = w_tap.astype(jnp.float32)
    n_elem = jnp.float32(8 * N * DHW)
    sum_row = w_tap @ sx
    sumsq_row = jnp.einsum("ri,ij,rj->r", w_tap, sxx, w_tap)
    mean_c = sum_row.reshape(8, Cout).sum(axis=0) / n_elem
    var_c = sumsq_row.reshape(8, Cout).sum(axis=0) / n_elem - mean_c * mean_c
    var_c = jnp.maximum(var_c, 0.0)
    scale_c = gamma.astype(jnp.float32) * jax.lax.rsqrt(var_c + _EPS)
    shift_c = beta.astype(jnp.float32) - mean_c * scale_c

    w4 = jnp.transpose(weight, (2, 1, 3, 4, 0)).reshape(2 * Cout, 4 * Cin)
    w4 = w4.astype(jnp.float32) * jnp.tile(scale_c, 2)[:, None]
    b4 = jnp.tile(shift_c, 2).reshape(2 * Cout, 1)

    F = 4 * HW
    m_idx = jnp.arange(HW)
    lane = 4 * W * (m_idx // W) + 2 * (m_idx % W)
    e0 = (jax.nn.one_hot(lane, F, dtype=jnp.float32))

    dt = 4 if D % 4 == 0 else (2 if D % 2 == 0 else 1)
    J = D // dt
    S = dt * HW

    def fused_kernel(w_ref, b_ref, e_ref, x_ref, o_ref, x4_ref):
        for dl in range(dt):
            s = jnp.dot(x_ref[:, dl * HW:(dl + 1) * HW], e_ref[...],
                        preferred_element_type=jnp.float32)
            x4_ref[0 * Cin:1 * Cin, dl * F:(dl + 1) * F] = s
            x4_ref[1 * Cin:2 * Cin, dl * F:(dl + 1) * F] = pltpu.roll(s, 1, 1)
            x4_ref[2 * Cin:3 * Cin, dl * F:(dl + 1) * F] = pltpu.roll(s, 2 * W, 1)
            x4_ref[3 * Cin:4 * Cin, dl * F:(dl + 1) * F] = pltpu.roll(s, 2 * W + 1, 1)
        y4 = jnp.maximum(
            jnp.dot(w_ref[...], x4_ref[...],
                    preferred_element_type=jnp.float32) + b_ref[...],
            0.0)
        for dl in range(dt):
            for kd in range(2):
                o_ref[:, (2 * dl + kd) * F:(2 * dl + kd + 1) * F] = (
                    y4[kd * Cout:(kd + 1) * Cout, dl * F:(dl + 1) * F])

    out3 = pl.pallas_call(
        fused_kernel,
        out_shape=jax.ShapeDtypeStruct((N, Cout, 8 * DHW), jnp.float32),
        grid=(N, J),
        in_specs=[
            pl.BlockSpec((2 * Cout, 4 * Cin), lambda n, j: (0, 0)),
            pl.BlockSpec((2 * Cout, 1), lambda n, j: (0, 0)),
            pl.BlockSpec((HW, F), lambda n, j: (0, 0)),
            pl.BlockSpec((pl.Squeezed(), Cin, S), lambda n, j: (n, 0, j)),
        ],
        out_specs=pl.BlockSpec((pl.Squeezed(), Cout, 8 * S),
                               lambda n, j: (n, 0, j)),
        scratch_shapes=[pltpu.VMEM((4 * Cin, dt * F), jnp.float32)],
        compiler_params=pltpu.CompilerParams(
            dimension_semantics=("parallel", "parallel"),
            vmem_limit_bytes=60 << 20),
    )(w4, b4, e0, x3)

    return out3.reshape(N, Cout, 2 * D, 2 * H, 2 * W)


# speedup vs baseline: 4.3715x; 1.0419x over previous
import jax
import jax.numpy as jnp
from jax.experimental import pallas as pl
from jax.experimental.pallas import tpu as pltpu

_EPS = 1e-5


def kernel(x, weight, bias, gamma, beta):
    del bias

    N, Cin, D, H, W = x.shape
    Cout = weight.shape[1]
    HW = H * W
    DHW = D * HW
    R = 8 * Cout

    x3 = x.reshape(N, Cin, DHW).astype(jnp.float32)

    P = 2 if N % 2 == 0 else 1
    NP = N // P

    def stats_kernel(x_ref, sxx_ref, sx_ref):
        @pl.when(pl.program_id(1) == 0)
        def _init():
            sxx_ref[...] = jnp.zeros_like(sxx_ref)
            sx_ref[...] = jnp.zeros_like(sx_ref)

        xb = x_ref[...]
        sxx_ref[...] += jax.lax.dot_general(
            xb, xb, (((1,), (1,)), ((), ())),
            preferred_element_type=jnp.float32)
        sx_ref[...] += jnp.sum(xb, axis=1, keepdims=True)

    psxx, psx = pl.pallas_call(
        stats_kernel,
        out_shape=(jax.ShapeDtypeStruct((P, Cin, Cin), jnp.float32),
                   jax.ShapeDtypeStruct((P, Cin, 1), jnp.float32)),
        grid=(P, NP),
        in_specs=[pl.BlockSpec((pl.Squeezed(), Cin, DHW),
                               lambda p, i: (p * NP + i, 0, 0))],
        out_specs=(pl.BlockSpec((pl.Squeezed(), Cin, Cin),
                                lambda p, i: (p, 0, 0)),
                   pl.BlockSpec((pl.Squeezed(), Cin, 1),
                                lambda p, i: (p, 0, 0))),
        compiler_params=pltpu.CompilerParams(
            dimension_semantics=("parallel", "arbitrary")),
    )(x3)
    sxx = psxx.sum(axis=0)
    sx = psx.sum(axis=0)[:, 0]

    w_tap = jnp.transpose(weight, (2, 3, 4, 1, 0)).reshape(R, Cin)
    w_tap = w_tap.astype(jnp.float32)
    n_elem = jnp.float32(8 * N * DHW)
    sum_row = w_tap @ sx
    sumsq_row = jnp.einsum("ri,ij,rj->r", w_tap, sxx, w_tap)
    mean_c = sum_row.reshape(8, Cout).sum(axis=0) / n_elem
    var_c = sumsq_row.reshape(8, Cout).sum(axis=0) / n_elem - mean_c * mean_c
    var_c = jnp.maximum(var_c, 0.0)
    scale_c = gamma.astype(jnp.float32) * jax.lax.rsqrt(var_c + _EPS)
    shift_c = beta.astype(jnp.float32) - mean_c * scale_c

    w4 = jnp.transpose(weight, (2, 1, 3, 4, 0)).reshape(2 * Cout, 4 * Cin)
    w4 = w4.astype(jnp.float32) * jnp.tile(scale_c, 2)[:, None]
    b4 = jnp.tile(shift_c, 2).reshape(2 * Cout, 1)

    F = 4 * HW
    m_idx = jnp.arange(HW)
    lane = 4 * W * (m_idx // W) + 2 * (m_idx % W)
    e0 = (jax.nn.one_hot(lane, F, dtype=jnp.float32))

    dt = 8 if D % 8 == 0 else (2 if D % 2 == 0 else 1)
    J = D // dt
    S = dt * HW

    def fused_kernel(w_ref, b_ref, e_ref, x_ref, o_ref, x4_ref):
        for dl in range(dt):
            s = jnp.dot(x_ref[:, dl * HW:(dl + 1) * HW], e_ref[...],
                        preferred_element_type=jnp.float32)
            x4_ref[0 * Cin:1 * Cin, dl * F:(dl + 1) * F] = s
            x4_ref[1 * Cin:2 * Cin, dl * F:(dl + 1) * F] = pltpu.roll(s, 1, 1)
            x4_ref[2 * Cin:3 * Cin, dl * F:(dl + 1) * F] = pltpu.roll(s, 2 * W, 1)
            x4_ref[3 * Cin:4 * Cin, dl * F:(dl + 1) * F] = pltpu.roll(s, 2 * W + 1, 1)
        y4 = jnp.maximum(
            jnp.dot(w_ref[...], x4_ref[...],
                    preferred_element_type=jnp.float32) + b_ref[...],
            0.0)
        for dl in range(dt):
            for kd in range(2):
                o_ref[:, (2 * dl + kd) * F:(2 * dl + kd + 1) * F] = (
                    y4[kd * Cout:(kd + 1) * Cout, dl * F:(dl + 1) * F])

    out3 = pl.pallas_call(
        fused_kernel,
        out_shape=jax.ShapeDtypeStruct((N, Cout, 8 * DHW), jnp.float32),
        grid=(N, J),
        in_specs=[
            pl.BlockSpec((2 * Cout, 4 * Cin), lambda n, j: (0, 0)),
            pl.BlockSpec((2 * Cout, 1), lambda n, j: (0, 0)),
            pl.BlockSpec((HW, F), lambda n, j: (0, 0)),
            pl.BlockSpec((pl.Squeezed(), Cin, S), lambda n, j: (n, 0, j)),
        ],
        out_specs=pl.BlockSpec((pl.Squeezed(), Cout, 8 * S),
                               lambda n, j: (n, 0, j)),
        scratch_shapes=[pltpu.VMEM((4 * Cin, dt * F), jnp.float32)],
        compiler_params=pltpu.CompilerParams(
            dimension_semantics=("parallel", "parallel"),
            vmem_limit_bytes=60 << 20),
    )(w4, b4, e0, x3)

    return out3.reshape(N, Cout, 2 * D, 2 * H, 2 * W)
